# Initial kernel scaffold; baseline (speedup 1.0000x reference)
import jax
import jax.numpy as jnp
from jax.experimental import pallas as pl


def kernel(Y, X, y_idx, x_idx):
    raise NotImplementedError("write your pallas kernel here")


# speedup vs baseline: 1.9692x; 1.9692x over previous
import jax
import jax.numpy as jnp
from jax.experimental import pallas as pl
from jax.experimental.pallas import tpu as pltpu

_B, _H, _W = 16, 2048, 2048
_K = 64
_RB = 256
_NRB = _H // _RB


def _body(y_s, x_s, lo_s, hi_s, X_s, Yb_ref, out_ref):
    b = pl.program_id(0)
    rb = pl.program_id(1)
    out_ref[...] = Yb_ref[...]
    lane = jax.lax.broadcasted_iota(jnp.int32, (1, _W), 1)

    def upd(k, carry):
        y = y_s[k]
        col = x_s[k]
        val = X_s[b, k]
        local = y - rb * _RB
        row = Yb_ref[0, pl.ds(local, 1), :]
        out_ref[0, pl.ds(local, 1), :] = row + jnp.where(lane == col, val, 0.0)
        return carry

    jax.lax.fori_loop(lo_s[rb], hi_s[rb], upd, 0)


def kernel(Y, X, y_idx, x_idx):
    edges = jnp.arange(_NRB, dtype=jnp.int32) * _RB
    lo = jnp.searchsorted(y_idx, edges).astype(jnp.int32)
    hi = jnp.searchsorted(y_idx, edges + _RB).astype(jnp.int32)
    out = pl.pallas_call(
        _body,
        grid_spec=pltpu.PrefetchScalarGridSpec(
            num_scalar_prefetch=5,
            grid=(_B, _NRB),
            in_specs=[
                pl.BlockSpec((1, _RB, _W), lambda b, rb, *_: (b, rb, 0)),
            ],
            out_specs=pl.BlockSpec((1, _RB, _W), lambda b, rb, *_: (b, rb, 0)),
        ),
        out_shape=jax.ShapeDtypeStruct((_B, _H, _W), jnp.float32),
    )(y_idx, x_idx, lo, hi, X, Y)
    return out
